# baseline (device time: 9792 ns/iter reference)
import jax
import jax.numpy as jnp
from jax import lax
from jax.experimental import pallas as pl
from jax.experimental.pallas import tpu as pltpu

N_DEV = 16


def kernel(x):
    m_per, n = x.shape

    def body(
        x_hbm, out_ref, xv_ref, own_ref, comm_ref, copy_sem, send_sems, recv_sems
    ):
        my = lax.axis_index("i")

        comm_ref[...] = jnp.full((N_DEV, 1, n), -jnp.inf, out_ref.dtype)
        bar = pltpu.get_barrier_semaphore()
        for off in range(1, N_DEV):
            pl.semaphore_signal(
                bar,
                inc=1,
                device_id=((my + off) % N_DEV,),
                device_id_type=pl.DeviceIdType.MESH,
            )

        cp = pltpu.make_async_copy(x_hbm, xv_ref, copy_sem)
        cp.start()
        cp.wait()
        lmax = jnp.max(xv_ref[...], axis=0, keepdims=True)
        own_ref[...] = lmax

        pl.semaphore_wait(bar, N_DEV - 1)

        sends = []
        for off in range(1, N_DEV):
            tgt = (my + off) % N_DEV
            rdma = pltpu.make_async_remote_copy(
                src_ref=own_ref,
                dst_ref=comm_ref.at[my],
                send_sem=send_sems.at[off],
                recv_sem=recv_sems.at[off],
                device_id=(tgt,),
                device_id_type=pl.DeviceIdType.MESH,
            )
            rdma.start()
            sends.append(rdma)

        for off in range(1, N_DEV):
            src = (my - off) % N_DEV
            recv = pltpu.make_async_remote_copy(
                src_ref=own_ref,
                dst_ref=comm_ref.at[src],
                send_sem=send_sems.at[0],
                recv_sem=recv_sems.at[off],
                device_id=(src,),
                device_id_type=pl.DeviceIdType.MESH,
            )
            recv.wait_recv()

        peers = jnp.max(comm_ref[...].reshape(N_DEV, n), axis=0, keepdims=True)
        out_ref[...] = jnp.maximum(lmax, peers)

        for rdma in sends:
            rdma.wait_send()

    return pl.pallas_call(
        body,
        out_shape=jax.ShapeDtypeStruct((1, n), x.dtype),
        in_specs=[pl.BlockSpec(memory_space=pltpu.MemorySpace.HBM)],
        out_specs=pl.BlockSpec(memory_space=pltpu.VMEM),
        scratch_shapes=[
            pltpu.VMEM((m_per, n), x.dtype),
            pltpu.VMEM((1, n), x.dtype),
            pltpu.VMEM((N_DEV, 1, n), x.dtype),
            pltpu.SemaphoreType.DMA,
            pltpu.SemaphoreType.DMA((N_DEV,)),
            pltpu.SemaphoreType.DMA((N_DEV,)),
        ],
        compiler_params=pltpu.CompilerParams(collective_id=0),
    )(x)


# device time: 9773 ns/iter; 1.0019x vs baseline; 1.0019x over previous
import jax
import jax.numpy as jnp
from jax import lax
from jax.experimental import pallas as pl
from jax.experimental.pallas import tpu as pltpu

N_DEV = 16


def kernel(x):
    m_per, n = x.shape

    def body(x_ref, out_ref, own_ref, comm_ref, send_sems, recv_sems):
        my = lax.axis_index("i")

        comm_ref[...] = jnp.full((N_DEV, 1, n), -jnp.inf, x_ref.dtype)
        bar = pltpu.get_barrier_semaphore()
        for off in range(1, N_DEV):
            pl.semaphore_signal(
                bar,
                inc=1,
                device_id=((my + off) % N_DEV,),
                device_id_type=pl.DeviceIdType.MESH,
            )

        lmax = jnp.max(x_ref[...], axis=0, keepdims=True)
        own_ref[...] = lmax

        pl.semaphore_wait(bar, N_DEV - 1)

        sends = []
        for off in range(1, N_DEV):
            tgt = (my + off) % N_DEV
            rdma = pltpu.make_async_remote_copy(
                src_ref=own_ref,
                dst_ref=comm_ref.at[my],
                send_sem=send_sems.at[off],
                recv_sem=recv_sems.at[off],
                device_id=(tgt,),
                device_id_type=pl.DeviceIdType.MESH,
            )
            rdma.start()
            sends.append(rdma)

        for off in range(1, N_DEV):
            src = (my - off) % N_DEV
            recv = pltpu.make_async_remote_copy(
                src_ref=own_ref,
                dst_ref=comm_ref.at[src],
                send_sem=send_sems.at[0],
                recv_sem=recv_sems.at[off],
                device_id=(src,),
                device_id_type=pl.DeviceIdType.MESH,
            )
            recv.wait_recv()

        peers = jnp.max(comm_ref[...].reshape(N_DEV, n), axis=0, keepdims=True)
        out_ref[...] = jnp.maximum(lmax, peers)

        for rdma in sends:
            rdma.wait_send()

    return pl.pallas_call(
        body,
        out_shape=jax.ShapeDtypeStruct((1, n), x.dtype),
        in_specs=[pl.BlockSpec(memory_space=pltpu.VMEM)],
        out_specs=pl.BlockSpec(memory_space=pltpu.VMEM),
        scratch_shapes=[
            pltpu.VMEM((1, n), x.dtype),
            pltpu.VMEM((N_DEV, 1, n), x.dtype),
            pltpu.SemaphoreType.DMA((N_DEV,)),
            pltpu.SemaphoreType.DMA((N_DEV,)),
        ],
        compiler_params=pltpu.CompilerParams(collective_id=0),
    )(x)
